# Initial kernel scaffold; baseline (speedup 1.0000x reference)
import jax
import jax.numpy as jnp
from jax.experimental import pallas as pl


def kernel(locations, table):
    raise NotImplementedError("write your pallas kernel here")


# speedup vs baseline: 4.0868x; 4.0868x over previous
import functools

import jax
import jax.numpy as jnp
from jax import lax
from jax.experimental import pallas as pl
from jax.experimental.pallas import tpu as pltpu
from jax.experimental.pallas import tpu_sc as plsc

EMBED = 64
NC = 2
NS = 16
NW = NC * NS
GROUP = 128


@functools.partial(jax.jit, static_argnames=("n_rows",))
def _sc_gather(table, idx, n_rows):
    rows_per_w = n_rows // NW
    groups_per_w = rows_per_w // GROUP
    mesh = plsc.VectorSubcoreMesh(core_axis_name="c", subcore_axis_name="s")

    @functools.partial(
        pl.kernel,
        mesh=mesh,
        out_type=jax.ShapeDtypeStruct((n_rows, EMBED), jnp.float32),
        scratch_types=[
            pltpu.VMEM((rows_per_w,), jnp.int32),
            pltpu.VMEM((GROUP, EMBED), jnp.float32),
            pltpu.SemaphoreType.DMA,
        ],
        compiler_params=pltpu.CompilerParams(use_tc_tiling_on_sc=False),
    )
    def k(table_hbm, idx_hbm, out_hbm, idx_v, rows_v, sem):
        wid = lax.axis_index("s") * NC + lax.axis_index("c")
        base = wid * rows_per_w
        pltpu.sync_copy(idx_hbm.at[pl.ds(base, rows_per_w)], idx_v)

        def body(j, _):
            pltpu.async_copy(
                table_hbm.at[idx_v.at[pl.ds(j * GROUP, GROUP)]], rows_v, sem
            ).wait()
            pltpu.sync_copy(
                rows_v, out_hbm.at[pl.ds(base + j * GROUP, GROUP)]
            )
            return 0

        lax.fori_loop(0, groups_per_w, body, 0)

    return k(table, idx)


def kernel(locations, table):
    n_rows = locations.size
    idx = locations.reshape(-1).astype(jnp.int32)
    out = _sc_gather(table, idx, n_rows)
    return out.reshape(locations.shape + (EMBED,))


# speedup vs baseline: 4.5795x; 1.1206x over previous
import functools

import jax
import jax.numpy as jnp
from jax import lax
from jax.experimental import pallas as pl
from jax.experimental.pallas import tpu as pltpu
from jax.experimental.pallas import tpu_sc as plsc

EMBED = 64
NC = 2
NS = 16
NW = NC * NS
GROUP = 128
NBUF = 5


@functools.partial(jax.jit, static_argnames=("n_rows",))
def _sc_gather(table, idx, n_rows):
    rows_per_w = n_rows // NW
    groups_per_w = rows_per_w // GROUP
    n_outer = groups_per_w // NBUF
    mesh = plsc.VectorSubcoreMesh(core_axis_name="c", subcore_axis_name="s")

    @functools.partial(
        pl.kernel,
        mesh=mesh,
        out_type=jax.ShapeDtypeStruct((n_rows, EMBED), jnp.float32),
        scratch_types=[
            pltpu.VMEM((rows_per_w,), jnp.int32),
            *[pltpu.VMEM((GROUP, EMBED), jnp.float32) for _ in range(NBUF)],
            pltpu.SemaphoreType.DMA,
            pltpu.SemaphoreType.DMA,
        ],
        compiler_params=pltpu.CompilerParams(use_tc_tiling_on_sc=False),
    )
    def k(table_hbm, idx_hbm, out_hbm, idx_v, *bufs_and_sems):
        bufs = bufs_and_sems[:NBUF]
        sem_g, sem_w = bufs_and_sems[NBUF:]
        wid = lax.axis_index("s") * NC + lax.axis_index("c")
        base = wid * rows_per_w
        pltpu.sync_copy(idx_hbm.at[pl.ds(base, rows_per_w)], idx_v)

        def wait_one_write():
            pltpu.make_async_copy(
                bufs[0], out_hbm.at[pl.ds(base, GROUP)], sem_w
            ).wait()

        def start_write(buf, s):
            pltpu.make_async_copy(
                buf, out_hbm.at[pl.ds(base + s * GROUP, GROUP)], sem_w
            ).start()

        def outer(g, _):
            descs = []
            for b in range(NBUF):
                s = g * NBUF + b

                @pl.when(g >= 1)
                def _():
                    wait_one_write()

                desc = pltpu.make_async_copy(
                    table_hbm.at[idx_v.at[pl.ds(s * GROUP, GROUP)]],
                    bufs[b],
                    sem_g,
                )
                desc.start()
                descs.append(desc)
                if b >= 1:
                    descs[b - 1].wait()
                    start_write(bufs[b - 1], s - 1)
            descs[NBUF - 1].wait()
            start_write(bufs[NBUF - 1], g * NBUF + NBUF - 1)
            return 0

        lax.fori_loop(0, n_outer, outer, 0)
        for _ in range(NBUF):
            wait_one_write()

    return k(table, idx)


def kernel(locations, table):
    n_rows = locations.size
    idx = locations.reshape(-1).astype(jnp.int32)
    out = _sc_gather(table, idx, n_rows)
    return out.reshape(locations.shape + (EMBED,))


# speedup vs baseline: 4.6575x; 1.0170x over previous
import functools

import jax
import jax.numpy as jnp
from jax import lax
from jax.experimental import pallas as pl
from jax.experimental.pallas import tpu as pltpu
from jax.experimental.pallas import tpu_sc as plsc

EMBED = 64
NC = 2
NS = 16
NW = NC * NS
GROUP = 128
NBUF = 10


@functools.partial(jax.jit, static_argnames=("n_rows",))
def _sc_gather(table, idx, n_rows):
    rows_per_w = n_rows // NW
    groups_per_w = rows_per_w // GROUP
    n_outer = groups_per_w // NBUF
    mesh = plsc.VectorSubcoreMesh(core_axis_name="c", subcore_axis_name="s")

    @functools.partial(
        pl.kernel,
        mesh=mesh,
        out_type=jax.ShapeDtypeStruct((n_rows, EMBED), jnp.float32),
        scratch_types=[
            pltpu.VMEM((rows_per_w,), jnp.int32),
            *[pltpu.VMEM((GROUP, EMBED), jnp.float32) for _ in range(NBUF)],
            pltpu.SemaphoreType.DMA,
            pltpu.SemaphoreType.DMA,
        ],
        compiler_params=pltpu.CompilerParams(use_tc_tiling_on_sc=False),
    )
    def k(table_hbm, idx_hbm, out_hbm, idx_v, *bufs_and_sems):
        bufs = bufs_and_sems[:NBUF]
        sem_g, sem_w = bufs_and_sems[NBUF:]
        wid = lax.axis_index("s") * NC + lax.axis_index("c")
        base = wid * rows_per_w
        pltpu.sync_copy(idx_hbm.at[pl.ds(base, rows_per_w)], idx_v)

        def wait_one_write():
            pltpu.make_async_copy(
                bufs[0], out_hbm.at[pl.ds(base, GROUP)], sem_w
            ).wait()

        def start_write(buf, s):
            pltpu.make_async_copy(
                buf, out_hbm.at[pl.ds(base + s * GROUP, GROUP)], sem_w
            ).start()

        def outer(g, _):
            descs = []
            for b in range(NBUF):
                s = g * NBUF + b

                @pl.when(g >= 1)
                def _():
                    wait_one_write()

                desc = pltpu.make_async_copy(
                    table_hbm.at[idx_v.at[pl.ds(s * GROUP, GROUP)]],
                    bufs[b],
                    sem_g,
                )
                desc.start()
                descs.append(desc)
            for b in range(NBUF):
                descs[b].wait()
                start_write(bufs[b], g * NBUF + b)
            return 0

        lax.fori_loop(0, n_outer, outer, 0)
        for _ in range(NBUF):
            wait_one_write()

    return k(table, idx)


def kernel(locations, table):
    n_rows = locations.size
    idx = locations.reshape(-1).astype(jnp.int32)
    out = _sc_gather(table, idx, n_rows)
    return out.reshape(locations.shape + (EMBED,))


# speedup vs baseline: 4.6719x; 1.0031x over previous
import functools

import jax
import jax.numpy as jnp
from jax import lax
from jax.experimental import pallas as pl
from jax.experimental.pallas import tpu as pltpu
from jax.experimental.pallas import tpu_sc as plsc

EMBED = 64
NC = 2
NS = 16
NW = NC * NS
NB = 8
NBUF = 2


@functools.partial(jax.jit, static_argnames=("batch", "hist"))
def _sc_gather(table, locations, batch, hist):
    b_per_w = batch // NW
    n_steps = b_per_w // NB
    n_outer = n_steps // NBUF
    mesh = plsc.VectorSubcoreMesh(core_axis_name="c", subcore_axis_name="s")

    @functools.partial(
        pl.kernel,
        mesh=mesh,
        out_type=jax.ShapeDtypeStruct((batch, hist, EMBED), jnp.float32),
        scratch_types=[
            pltpu.VMEM((b_per_w, hist), jnp.int32),
            *[pltpu.VMEM((NB, hist, EMBED), jnp.float32) for _ in range(NBUF)],
            pltpu.SemaphoreType.DMA,
            pltpu.SemaphoreType.DMA,
        ],
        compiler_params=pltpu.CompilerParams(use_tc_tiling_on_sc=False),
    )
    def k(table_hbm, idx_hbm, out_hbm, idx_v, *bufs_and_sems):
        bufs = bufs_and_sems[:NBUF]
        sem_g, sem_w = bufs_and_sems[NBUF:]
        wid = lax.axis_index("s") * NC + lax.axis_index("c")
        base = wid * b_per_w
        pltpu.sync_copy(idx_hbm.at[pl.ds(base, b_per_w)], idx_v)

        def wait_one_write():
            pltpu.make_async_copy(
                bufs[0], out_hbm.at[pl.ds(base, NB)], sem_w
            ).wait()

        def outer(g, _):
            descs = []
            for b in range(NBUF):
                t = g * NBUF + b

                @pl.when(g >= 1)
                def _():
                    wait_one_write()

                for j in range(NB):
                    desc = pltpu.make_async_copy(
                        table_hbm.at[idx_v.at[t * NB + j]],
                        bufs[b].at[j],
                        sem_g,
                    )
                    desc.start()
                    descs.append(desc)
            for b in range(NBUF):
                t = g * NBUF + b
                for j in range(NB):
                    descs[b * NB + j].wait()
                pltpu.make_async_copy(
                    bufs[b], out_hbm.at[pl.ds(base + t * NB, NB)], sem_w
                ).start()
            return 0

        lax.fori_loop(0, n_outer, outer, 0)
        for _ in range(NBUF):
            wait_one_write()

    return k(table, locations)


def kernel(locations, table):
    batch, hist = locations.shape
    return _sc_gather(table, locations, batch, hist)
